# baseline (speedup 1.0000x reference)
import functools

import jax
import jax.numpy as jnp
from jax import lax
from jax.experimental import pallas as pl
from jax.experimental.pallas import tpu as pltpu
from jax.experimental.pallas import tpu_sc as plsc

B = 16384
D = 32
NC = 2
NS = 16
NW = NC * NS
BPW = B // NW
CHUNK = 128
NCHUNK = BPW // CHUNK
L = 16


def _mf_body(users_hbm, items_hbm, ue_hbm, ie_hbm, ub_hbm, ib_hbm, out_hbm,
             uidx, iidx, urows, irows, ubias, ibias, ph, outv, sem):
    wid = lax.axis_index("s") * NC + lax.axis_index("c")
    base = pl.multiple_of(wid * BPW, BPW)

    for ch in range(NCHUNK):
        pltpu.sync_copy(users_hbm.at[pl.ds(base + ch * CHUNK, CHUNK)], uidx.at[ch])
        pltpu.sync_copy(items_hbm.at[pl.ds(base + ch * CHUNK, CHUNK)], iidx.at[ch])

    copies = []
    for ch in range(NCHUNK):
        sl = pl.ds(ch * CHUNK, CHUNK)
        copies.append(pltpu.async_copy(ue_hbm.at[uidx.at[ch]], urows.at[sl], sem))
        copies.append(pltpu.async_copy(ie_hbm.at[iidx.at[ch]], irows.at[sl], sem))
        copies.append(pltpu.async_copy(ub_hbm.at[uidx.at[ch]], ubias.at[sl], sem))
        copies.append(pltpu.async_copy(ib_hbm.at[iidx.at[ch]], ibias.at[sl], sem))
    for c in copies:
        c.wait()

    iota = lax.broadcasted_iota(jnp.int32, (L,), 0)
    colbase = iota * L

    def group_body(g, _):
        gb = pl.multiple_of(g * L, L)
        for i in range(L):
            u0 = urows[gb + i, pl.ds(0, L)]
            u1 = urows[gb + i, pl.ds(L, L)]
            v0 = irows[gb + i, pl.ds(0, L)]
            v1 = irows[gb + i, pl.ds(L, L)]
            ph[pl.ds(i * L, L)] = u0 * v0 + u1 * v1
        acc = ubias[pl.ds(gb, L)] + ibias[pl.ds(gb, L)]
        for j in range(L):
            acc = acc + plsc.load_gather(ph, [colbase + j])
        outv[pl.ds(gb, L)] = acc
        return 0

    lax.fori_loop(0, BPW // L, group_body, 0)
    pltpu.sync_copy(outv, out_hbm.at[pl.ds(base, BPW)])


@functools.partial(
    pl.kernel,
    out_type=jax.ShapeDtypeStruct((B,), jnp.float32),
    mesh=plsc.VectorSubcoreMesh(core_axis_name="c", subcore_axis_name="s"),
    compiler_params=pltpu.CompilerParams(
        needs_layout_passes=False, use_tc_tiling_on_sc=False),
    scratch_types=[
        pltpu.VMEM((NCHUNK, CHUNK), jnp.int32),
        pltpu.VMEM((NCHUNK, CHUNK), jnp.int32),
        pltpu.VMEM((BPW, D), jnp.float32),
        pltpu.VMEM((BPW, D), jnp.float32),
        pltpu.VMEM((BPW,), jnp.float32),
        pltpu.VMEM((BPW,), jnp.float32),
        pltpu.VMEM((L * L,), jnp.float32),
        pltpu.VMEM((BPW,), jnp.float32),
        pltpu.SemaphoreType.DMA,
    ],
)
def _mf_kernel(users_hbm, items_hbm, ue_hbm, ie_hbm, ub_hbm, ib_hbm, out_hbm,
               uidx, iidx, urows, irows, ubias, ibias, ph, outv, sem):
    _mf_body(users_hbm, items_hbm, ue_hbm, ie_hbm, ub_hbm, ib_hbm, out_hbm,
             uidx, iidx, urows, irows, ubias, ibias, ph, outv, sem)


def kernel(users, items, user_emb, item_emb, user_bias, item_bias):
    ub = user_bias.reshape(-1)
    ib = item_bias.reshape(-1)
    return _mf_kernel(users.astype(jnp.int32), items.astype(jnp.int32),
                      user_emb, item_emb, ub, ib)


# speedup vs baseline: 2.5365x; 2.5365x over previous
import functools

import jax
import jax.numpy as jnp
from jax import lax
from jax.experimental import pallas as pl
from jax.experimental.pallas import tpu as pltpu
from jax.experimental.pallas import tpu_sc as plsc

B = 16384
D = 32
V = 1000001
NC = 2
NS = 16
NW = NC * NS
BPW = B // NW
L = 16
FLIGHT = 8
BLK = 128


def _mf_body(users_hbm, items_hbm, ue_t, ie_t, ub_hbm, ib_hbm, out_hbm,
             uidx, iidx, jidx, ublk, iblk, urows, irows,
             ubias, ibias, ph, outv, semg, semb):
    wid = lax.axis_index("s") * NC + lax.axis_index("c")
    base = pl.multiple_of(wid * BPW, BPW)

    pltpu.sync_copy(users_hbm.at[pl.ds(base, BPW)], uidx.at[pl.ds(0, BPW)])
    pltpu.sync_copy(items_hbm.at[pl.ds(base, BPW)], iidx.at[pl.ds(0, BPW)])

    iota = lax.broadcasted_iota(jnp.int32, (L,), 0)

    jidx[pl.ds(0, L)] = iota
    jidx[pl.ds(L, L)] = iota + L

    bias_copies = []
    for c in range(BPW // BLK):
        sl = pl.ds(c * BLK, BLK)
        bias_copies.append(pltpu.async_copy(ub_hbm.at[uidx.at[sl]], ubias.at[sl], semb))
        bias_copies.append(pltpu.async_copy(ib_hbm.at[iidx.at[sl]], ibias.at[sl], semb))

    jhi = iota + L

    def flight_body(f, _):
        kbase = pl.multiple_of(f * FLIGHT, FLIGHT)
        iv_u = uidx[pl.ds(kbase, L)]
        iv_i = iidx[pl.ds(kbase, L)]
        copies = []
        for l in range(FLIGHT):
            cu = pl.multiple_of((iv_u[l] >> 7) * BLK, BLK)
            ci = pl.multiple_of((iv_i[l] >> 7) * BLK, BLK)
            copies.append(pltpu.async_copy(
                ue_t.at[jidx, pl.ds(cu, BLK)], ublk.at[l], semg))
            copies.append(pltpu.async_copy(
                ie_t.at[jidx, pl.ds(ci, BLK)], iblk.at[l], semg))
        for c in copies:
            c.wait()
        for l in range(FLIGHT):
            cu = jnp.broadcast_to(iv_u[l] & (BLK - 1), (L,))
            ci = jnp.broadcast_to(iv_i[l] & (BLK - 1), (L,))
            lv = jnp.broadcast_to(l, (L,))
            k32 = (kbase + l) * D
            urows[pl.ds(k32, L)] = plsc.load_gather(ublk, [lv, iota, cu])
            urows[pl.ds(k32 + L, L)] = plsc.load_gather(ublk, [lv, jhi, cu])
            irows[pl.ds(k32, L)] = plsc.load_gather(iblk, [lv, iota, ci])
            irows[pl.ds(k32 + L, L)] = plsc.load_gather(iblk, [lv, jhi, ci])
        return 0

    lax.fori_loop(0, BPW // FLIGHT, flight_body, 0)
    for c in bias_copies:
        c.wait()

    colbase = iota * L

    def group_body(g, _):
        gb = pl.multiple_of(g * L, L)
        for i in range(L):
            r = (gb + i) * D
            u0 = urows[pl.ds(r, L)]
            u1 = urows[pl.ds(r + L, L)]
            v0 = irows[pl.ds(r, L)]
            v1 = irows[pl.ds(r + L, L)]
            ph[pl.ds(i * L, L)] = u0 * v0 + u1 * v1
        acc = ubias[pl.ds(gb, L)] + ibias[pl.ds(gb, L)]
        for j in range(L):
            acc = acc + plsc.load_gather(ph, [colbase + j])
        outv[pl.ds(gb, L)] = acc
        return 0

    lax.fori_loop(0, BPW // L, group_body, 0)
    pltpu.sync_copy(outv, out_hbm.at[pl.ds(base, BPW)])


@functools.partial(
    pl.kernel,
    out_type=jax.ShapeDtypeStruct((B,), jnp.float32),
    mesh=plsc.VectorSubcoreMesh(core_axis_name="c", subcore_axis_name="s"),
    compiler_params=pltpu.CompilerParams(
        needs_layout_passes=False,
        use_tc_tiling_on_sc=True,
        disable_bounds_checks=True,
    ),
    scratch_types=[
        pltpu.VMEM((BPW + L,), jnp.int32),
        pltpu.VMEM((BPW + L,), jnp.int32),
        pltpu.VMEM((D,), jnp.int32),
        pltpu.VMEM((FLIGHT, D, BLK), jnp.float32),
        pltpu.VMEM((FLIGHT, D, BLK), jnp.float32),
        pltpu.VMEM((BPW * D,), jnp.float32),
        pltpu.VMEM((BPW * D,), jnp.float32),
        pltpu.VMEM((BPW,), jnp.float32),
        pltpu.VMEM((BPW,), jnp.float32),
        pltpu.VMEM((L * L,), jnp.float32),
        pltpu.VMEM((BPW,), jnp.float32),
        pltpu.SemaphoreType.DMA,
        pltpu.SemaphoreType.DMA,
    ],
)
def _mf_kernel(users_hbm, items_hbm, ue_t, ie_t, ub_hbm, ib_hbm, out_hbm,
               uidx, iidx, jidx, ublk, iblk, urows, irows,
               ubias, ibias, ph, outv, semg, semb):
    _mf_body(users_hbm, items_hbm, ue_t, ie_t, ub_hbm, ib_hbm, out_hbm,
             uidx, iidx, jidx, ublk, iblk, urows, irows,
             ubias, ibias, ph, outv, semg, semb)


def kernel(users, items, user_emb, item_emb, user_bias, item_bias):
    return _mf_kernel(users.astype(jnp.int32), items.astype(jnp.int32),
                      user_emb.T, item_emb.T,
                      user_bias.reshape(-1), item_bias.reshape(-1))


# speedup vs baseline: 3.1280x; 1.2332x over previous
import functools

import jax
import jax.numpy as jnp
from jax import lax
from jax.experimental import pallas as pl
from jax.experimental.pallas import tpu as pltpu
from jax.experimental.pallas import tpu_sc as plsc

B = 16384
D = 32
V = 1000001
NC = 2
NS = 16
NW = NC * NS
BPW = B // NW
L = 16
FLIGHT = 8
BLK = 128


def _mf_body(users_hbm, items_hbm, ue_t, ie_t, ub_t, ib_t, out_hbm,
             uidx, iidx, jidx, ublk, iblk, ubb, ibb, urows, irows,
             ubias, ibias, ph, outv, semg, semb):
    wid = lax.axis_index("s") * NC + lax.axis_index("c")
    base = pl.multiple_of(wid * BPW, BPW)

    pltpu.sync_copy(users_hbm.at[pl.ds(base, BPW)], uidx.at[pl.ds(0, BPW)])
    pltpu.sync_copy(items_hbm.at[pl.ds(base, BPW)], iidx.at[pl.ds(0, BPW)])

    iota = lax.broadcasted_iota(jnp.int32, (L,), 0)

    jidx[pl.ds(0, L)] = iota
    jidx[pl.ds(L, L)] = iota + L

    jhi = iota + L
    zeros = jnp.broadcast_to(0, (L,))
    lane_mod = iota & (FLIGHT - 1)
    lane_lo = iota < FLIGHT

    def flight_body(f, _):
        kbase = pl.multiple_of(f * FLIGHT, FLIGHT)
        iv_u = uidx[pl.ds(kbase, L)]
        iv_i = iidx[pl.ds(kbase, L)]
        copies = []
        for l in range(FLIGHT):
            cu = pl.multiple_of((iv_u[l] >> 7) * BLK, BLK)
            ci = pl.multiple_of((iv_i[l] >> 7) * BLK, BLK)
            copies.append(pltpu.async_copy(
                ue_t.at[jidx, pl.ds(cu, BLK)], ublk.at[l], semg))
            copies.append(pltpu.async_copy(
                ie_t.at[jidx, pl.ds(ci, BLK)], iblk.at[l], semg))
            copies.append(pltpu.async_copy(
                ub_t.at[pl.ds(0, 1), pl.ds(cu, BLK)], ubb.at[l], semb))
            copies.append(pltpu.async_copy(
                ib_t.at[pl.ds(0, 1), pl.ds(ci, BLK)], ibb.at[l], semb))
        for c in copies:
            c.wait()
        ubv = plsc.load_gather(ubb, [lane_mod, zeros, iv_u & (BLK - 1)])
        ibv = plsc.load_gather(ibb, [lane_mod, zeros, iv_i & (BLK - 1)])
        plsc.store_scatter(ubias, [kbase + iota], ubv, mask=lane_lo)
        plsc.store_scatter(ibias, [kbase + iota], ibv, mask=lane_lo)
        for l in range(FLIGHT):
            cu = jnp.broadcast_to(iv_u[l] & (BLK - 1), (L,))
            ci = jnp.broadcast_to(iv_i[l] & (BLK - 1), (L,))
            lv = jnp.broadcast_to(l, (L,))
            k32 = (kbase + l) * D
            urows[pl.ds(k32, L)] = plsc.load_gather(ublk, [lv, iota, cu])
            urows[pl.ds(k32 + L, L)] = plsc.load_gather(ublk, [lv, jhi, cu])
            irows[pl.ds(k32, L)] = plsc.load_gather(iblk, [lv, iota, ci])
            irows[pl.ds(k32 + L, L)] = plsc.load_gather(iblk, [lv, jhi, ci])
        return 0

    lax.fori_loop(0, BPW // FLIGHT, flight_body, 0)

    colbase = iota * L

    def group_body(g, _):
        gb = pl.multiple_of(g * L, L)
        for i in range(L):
            r = (gb + i) * D
            u0 = urows[pl.ds(r, L)]
            u1 = urows[pl.ds(r + L, L)]
            v0 = irows[pl.ds(r, L)]
            v1 = irows[pl.ds(r + L, L)]
            ph[pl.ds(i * L, L)] = u0 * v0 + u1 * v1
        acc = ubias[pl.ds(gb, L)] + ibias[pl.ds(gb, L)]
        for j in range(L):
            acc = acc + plsc.load_gather(ph, [colbase + j])
        outv[pl.ds(gb, L)] = acc
        return 0

    lax.fori_loop(0, BPW // L, group_body, 0)
    pltpu.sync_copy(outv, out_hbm.at[pl.ds(base, BPW)])


@functools.partial(
    pl.kernel,
    out_type=jax.ShapeDtypeStruct((B,), jnp.float32),
    mesh=plsc.VectorSubcoreMesh(core_axis_name="c", subcore_axis_name="s"),
    compiler_params=pltpu.CompilerParams(
        needs_layout_passes=False,
        use_tc_tiling_on_sc=True,
        disable_bounds_checks=True,
    ),
    scratch_types=[
        pltpu.VMEM((BPW + L,), jnp.int32),
        pltpu.VMEM((BPW + L,), jnp.int32),
        pltpu.VMEM((D,), jnp.int32),
        pltpu.VMEM((FLIGHT, D, BLK), jnp.float32),
        pltpu.VMEM((FLIGHT, D, BLK), jnp.float32),
        pltpu.VMEM((FLIGHT, 1, BLK), jnp.float32),
        pltpu.VMEM((FLIGHT, 1, BLK), jnp.float32),
        pltpu.VMEM((BPW * D,), jnp.float32),
        pltpu.VMEM((BPW * D,), jnp.float32),
        pltpu.VMEM((BPW,), jnp.float32),
        pltpu.VMEM((BPW,), jnp.float32),
        pltpu.VMEM((L * L,), jnp.float32),
        pltpu.VMEM((BPW,), jnp.float32),
        pltpu.SemaphoreType.DMA,
        pltpu.SemaphoreType.DMA,
    ],
)
def _mf_kernel(users_hbm, items_hbm, ue_t, ie_t, ub_t, ib_t, out_hbm,
               uidx, iidx, jidx, ublk, iblk, ubb, ibb, urows, irows,
               ubias, ibias, ph, outv, semg, semb):
    _mf_body(users_hbm, items_hbm, ue_t, ie_t, ub_t, ib_t, out_hbm,
             uidx, iidx, jidx, ublk, iblk, ubb, ibb, urows, irows,
             ubias, ibias, ph, outv, semg, semb)


def kernel(users, items, user_emb, item_emb, user_bias, item_bias):
    return _mf_kernel(users.astype(jnp.int32), items.astype(jnp.int32),
                      user_emb.T, item_emb.T, user_bias.T, item_bias.T)


# speedup vs baseline: 3.4086x; 1.0897x over previous
import functools

import jax
import jax.numpy as jnp
from jax import lax
from jax.experimental import pallas as pl
from jax.experimental.pallas import tpu as pltpu
from jax.experimental.pallas import tpu_sc as plsc

B = 16384
D = 32
V = 1000001
NC = 2
NS = 16
NW = NC * NS
BPW = B // NW
L = 16
SF = 4
BLK = 128


def _mf_body(users_hbm, items_hbm, ue_t, ie_t, ub_t, ib_t, out_hbm,
             uidx, iidx, jidx, ublk, iblk, ubb, ibb, urows, irows,
             ubias, ibias, ph, outv, semg0, semg1, semb0, semb1):
    wid = lax.axis_index("s") * NC + lax.axis_index("c")
    base = pl.multiple_of(wid * BPW, BPW)

    pltpu.sync_copy(users_hbm.at[pl.ds(base, BPW)], uidx)
    pltpu.sync_copy(items_hbm.at[pl.ds(base, BPW)], iidx)

    iota = lax.broadcasted_iota(jnp.int32, (L,), 0)

    jidx[pl.ds(0, L)] = iota
    jidx[pl.ds(L, L)] = iota + L

    jhi = iota + L
    zeros = jnp.broadcast_to(0, (L,))
    lane_mod = iota & (SF - 1)

    def iter_body(g, _):
        kbase = pl.multiple_of(g * L, L)
        iv_u = uidx[pl.ds(kbase, L)]
        iv_i = iidx[pl.ds(kbase, L)]

        def fire(sub, bank):
            semg = semg0 if bank == 0 else semg1
            semb = semb0 if bank == 0 else semb1
            for l in range(SF):
                lane = sub * SF + l
                cu = pl.multiple_of((iv_u[lane] >> 7) * BLK, BLK)
                ci = pl.multiple_of((iv_i[lane] >> 7) * BLK, BLK)
                slot = bank * SF + l
                pltpu.async_copy(ue_t.at[jidx, pl.ds(cu, BLK)],
                                 ublk.at[slot], semg)
                pltpu.async_copy(ie_t.at[jidx, pl.ds(ci, BLK)],
                                 iblk.at[slot], semg)
                pltpu.async_copy(ub_t.at[pl.ds(0, 1), pl.ds(cu, BLK)],
                                 ubb.at[slot], semb)
                pltpu.async_copy(ib_t.at[pl.ds(0, 1), pl.ds(ci, BLK)],
                                 ibb.at[slot], semb)

        def drain_extract(sub, bank):
            semg = semg0 if bank == 0 else semg1
            semb = semb0 if bank == 0 else semb1
            for _ in range(SF):
                pltpu.make_async_copy(
                    ue_t.at[jidx, pl.ds(0, BLK)], ublk.at[0], semg).wait()
                pltpu.make_async_copy(
                    ie_t.at[jidx, pl.ds(0, BLK)], iblk.at[0], semg).wait()
                pltpu.make_async_copy(
                    ub_t.at[pl.ds(0, 1), pl.ds(0, BLK)], ubb.at[0], semb).wait()
                pltpu.make_async_copy(
                    ib_t.at[pl.ds(0, 1), pl.ds(0, BLK)], ibb.at[0], semb).wait()
            bank_row = jnp.broadcast_to(bank * SF, (L,)) + lane_mod
            ubv = plsc.load_gather(ubb, [bank_row, zeros, iv_u & (BLK - 1)])
            ibv = plsc.load_gather(ibb, [bank_row, zeros, iv_i & (BLK - 1)])
            sel = (iota >> 2) == sub
            kvec = kbase + sub * SF + lane_mod
            plsc.store_scatter(ubias, [kvec], ubv, mask=sel)
            plsc.store_scatter(ibias, [kvec], ibv, mask=sel)
            for l in range(SF):
                lane = sub * SF + l
                cu = jnp.broadcast_to(iv_u[lane] & (BLK - 1), (L,))
                ci = jnp.broadcast_to(iv_i[lane] & (BLK - 1), (L,))
                lv = jnp.broadcast_to(bank * SF + l, (L,))
                k32 = (kbase + lane) * D
                urows[pl.ds(k32, L)] = plsc.load_gather(ublk, [lv, iota, cu])
                urows[pl.ds(k32 + L, L)] = plsc.load_gather(ublk, [lv, jhi, cu])
                irows[pl.ds(k32, L)] = plsc.load_gather(iblk, [lv, iota, ci])
                irows[pl.ds(k32 + L, L)] = plsc.load_gather(iblk, [lv, jhi, ci])

        fire(0, 0)
        fire(1, 1)
        drain_extract(0, 0)
        fire(2, 0)
        drain_extract(1, 1)
        fire(3, 1)
        drain_extract(2, 0)
        drain_extract(3, 1)
        return 0

    lax.fori_loop(0, BPW // L, iter_body, 0)

    colbase = iota * L

    def group_body(g, _):
        gb = pl.multiple_of(g * L, L)
        for i in range(L):
            r = (gb + i) * D
            u0 = urows[pl.ds(r, L)]
            u1 = urows[pl.ds(r + L, L)]
            v0 = irows[pl.ds(r, L)]
            v1 = irows[pl.ds(r + L, L)]
            ph[pl.ds(i * L, L)] = u0 * v0 + u1 * v1
        acc = ubias[pl.ds(gb, L)] + ibias[pl.ds(gb, L)]
        for j in range(L):
            acc = acc + plsc.load_gather(ph, [colbase + j])
        outv[pl.ds(gb, L)] = acc
        return 0

    lax.fori_loop(0, BPW // L, group_body, 0)
    pltpu.sync_copy(outv, out_hbm.at[pl.ds(base, BPW)])


@functools.partial(
    pl.kernel,
    out_type=jax.ShapeDtypeStruct((B,), jnp.float32),
    mesh=plsc.VectorSubcoreMesh(core_axis_name="c", subcore_axis_name="s"),
    compiler_params=pltpu.CompilerParams(
        needs_layout_passes=False,
        use_tc_tiling_on_sc=True,
        disable_bounds_checks=True,
    ),
    scratch_types=[
        pltpu.VMEM((BPW,), jnp.int32),
        pltpu.VMEM((BPW,), jnp.int32),
        pltpu.VMEM((D,), jnp.int32),
        pltpu.VMEM((2 * SF, D, BLK), jnp.float32),
        pltpu.VMEM((2 * SF, D, BLK), jnp.float32),
        pltpu.VMEM((2 * SF, 1, BLK), jnp.float32),
        pltpu.VMEM((2 * SF, 1, BLK), jnp.float32),
        pltpu.VMEM((BPW * D,), jnp.float32),
        pltpu.VMEM((BPW * D,), jnp.float32),
        pltpu.VMEM((BPW,), jnp.float32),
        pltpu.VMEM((BPW,), jnp.float32),
        pltpu.VMEM((L * L,), jnp.float32),
        pltpu.VMEM((BPW,), jnp.float32),
        pltpu.SemaphoreType.DMA,
        pltpu.SemaphoreType.DMA,
        pltpu.SemaphoreType.DMA,
        pltpu.SemaphoreType.DMA,
    ],
)
def _mf_kernel(users_hbm, items_hbm, ue_t, ie_t, ub_t, ib_t, out_hbm,
               uidx, iidx, jidx, ublk, iblk, ubb, ibb, urows, irows,
               ubias, ibias, ph, outv, semg0, semg1, semb0, semb1):
    _mf_body(users_hbm, items_hbm, ue_t, ie_t, ub_t, ib_t, out_hbm,
             uidx, iidx, jidx, ublk, iblk, ubb, ibb, urows, irows,
             ubias, ibias, ph, outv, semg0, semg1, semb0, semb1)


def kernel(users, items, user_emb, item_emb, user_bias, item_bias):
    return _mf_kernel(users.astype(jnp.int32), items.astype(jnp.int32),
                      user_emb.T, item_emb.T, user_bias.T, item_bias.T)


# speedup vs baseline: 3.5105x; 1.0299x over previous
import functools

import jax
import jax.numpy as jnp
from jax import lax
from jax.experimental import pallas as pl
from jax.experimental.pallas import tpu as pltpu
from jax.experimental.pallas import tpu_sc as plsc

B = 16384
D = 32
V = 1000001
NC = 2
NS = 16
NW = NC * NS
BPW = B // NW
L = 16
SF = 4
BLK = 128


def _mf_body(users_hbm, items_hbm, ue_t, ie_t, ub_t, ib_t, out_hbm,
             uidx, iidx, jidx, ublk, iblk, ubb, ibb, phall,
             ubias, ibias, outv, semg0, semg1, semb0, semb1):
    wid = lax.axis_index("s") * NC + lax.axis_index("c")
    base = pl.multiple_of(wid * BPW, BPW)

    pltpu.sync_copy(users_hbm.at[pl.ds(base, BPW)], uidx)
    pltpu.sync_copy(items_hbm.at[pl.ds(base, BPW)], iidx)

    iota = lax.broadcasted_iota(jnp.int32, (L,), 0)

    jidx[pl.ds(0, L)] = iota
    jidx[pl.ds(L, L)] = iota + L

    jhi = iota + L
    zeros = jnp.broadcast_to(0, (L,))
    lane_mod = iota & (SF - 1)

    def iter_body(g, _):
        kbase = pl.multiple_of(g * L, L)
        iv_u = uidx[pl.ds(kbase, L)]
        iv_i = iidx[pl.ds(kbase, L)]

        def fire(sub, bank):
            semg = semg0 if bank == 0 else semg1
            semb = semb0 if bank == 0 else semb1
            for l in range(SF):
                lane = sub * SF + l
                cu = pl.multiple_of((iv_u[lane] >> 7) * BLK, BLK)
                ci = pl.multiple_of((iv_i[lane] >> 7) * BLK, BLK)
                slot = bank * SF + l
                pltpu.async_copy(ue_t.at[jidx, pl.ds(cu, BLK)],
                                 ublk.at[slot], semg)
                pltpu.async_copy(ie_t.at[jidx, pl.ds(ci, BLK)],
                                 iblk.at[slot], semg)
                pltpu.async_copy(ub_t.at[pl.ds(0, 1), pl.ds(cu, BLK)],
                                 ubb.at[slot], semb)
                pltpu.async_copy(ib_t.at[pl.ds(0, 1), pl.ds(ci, BLK)],
                                 ibb.at[slot], semb)

        def drain_extract(sub, bank):
            semg = semg0 if bank == 0 else semg1
            semb = semb0 if bank == 0 else semb1
            for _ in range(SF):
                pltpu.make_async_copy(
                    ue_t.at[jidx, pl.ds(0, BLK)], ublk.at[0], semg).wait()
                pltpu.make_async_copy(
                    ie_t.at[jidx, pl.ds(0, BLK)], iblk.at[0], semg).wait()
                pltpu.make_async_copy(
                    ub_t.at[pl.ds(0, 1), pl.ds(0, BLK)], ubb.at[0], semb).wait()
                pltpu.make_async_copy(
                    ib_t.at[pl.ds(0, 1), pl.ds(0, BLK)], ibb.at[0], semb).wait()
            bank_row = jnp.broadcast_to(bank * SF, (L,)) + lane_mod
            ubv = plsc.load_gather(ubb, [bank_row, zeros, iv_u & (BLK - 1)])
            ibv = plsc.load_gather(ibb, [bank_row, zeros, iv_i & (BLK - 1)])
            sel = (iota >> 2) == sub
            kvec = kbase + sub * SF + lane_mod
            plsc.store_scatter(ubias, [kvec], ubv, mask=sel)
            plsc.store_scatter(ibias, [kvec], ibv, mask=sel)
            for l in range(SF):
                lane = sub * SF + l
                cu = jnp.broadcast_to(iv_u[lane] & (BLK - 1), (L,))
                ci = jnp.broadcast_to(iv_i[lane] & (BLK - 1), (L,))
                lv = jnp.broadcast_to(bank * SF + l, (L,))
                ulo = plsc.load_gather(ublk, [lv, iota, cu])
                uhi = plsc.load_gather(ublk, [lv, jhi, cu])
                vlo = plsc.load_gather(iblk, [lv, iota, ci])
                vhi = plsc.load_gather(iblk, [lv, jhi, ci])
                phall[pl.ds((kbase + lane) * L, L)] = ulo * vlo + uhi * vhi

        fire(0, 0)
        fire(1, 1)
        drain_extract(0, 0)
        fire(2, 0)
        drain_extract(1, 1)
        fire(3, 1)
        drain_extract(2, 0)
        drain_extract(3, 1)
        return 0

    lax.fori_loop(0, BPW // L, iter_body, 0)

    colbase = iota * L

    def group_body(g, _):
        gb = pl.multiple_of(g * L, L)
        gidx = colbase + gb * L
        acc = ubias[pl.ds(gb, L)] + ibias[pl.ds(gb, L)]
        for j in range(L):
            acc = acc + plsc.load_gather(phall, [gidx + j])
        outv[pl.ds(gb, L)] = acc
        return 0

    lax.fori_loop(0, BPW // L, group_body, 0)
    pltpu.sync_copy(outv, out_hbm.at[pl.ds(base, BPW)])


@functools.partial(
    pl.kernel,
    out_type=jax.ShapeDtypeStruct((B,), jnp.float32),
    mesh=plsc.VectorSubcoreMesh(core_axis_name="c", subcore_axis_name="s"),
    compiler_params=pltpu.CompilerParams(
        needs_layout_passes=False,
        use_tc_tiling_on_sc=True,
        disable_bounds_checks=True,
    ),
    scratch_types=[
        pltpu.VMEM((BPW,), jnp.int32),
        pltpu.VMEM((BPW,), jnp.int32),
        pltpu.VMEM((D,), jnp.int32),
        pltpu.VMEM((2 * SF, D, BLK), jnp.float32),
        pltpu.VMEM((2 * SF, D, BLK), jnp.float32),
        pltpu.VMEM((2 * SF, 1, BLK), jnp.float32),
        pltpu.VMEM((2 * SF, 1, BLK), jnp.float32),
        pltpu.VMEM((BPW * L,), jnp.float32),
        pltpu.VMEM((BPW,), jnp.float32),
        pltpu.VMEM((BPW,), jnp.float32),
        pltpu.VMEM((BPW,), jnp.float32),
        pltpu.SemaphoreType.DMA,
        pltpu.SemaphoreType.DMA,
        pltpu.SemaphoreType.DMA,
        pltpu.SemaphoreType.DMA,
    ],
)
def _mf_kernel(users_hbm, items_hbm, ue_t, ie_t, ub_t, ib_t, out_hbm,
               uidx, iidx, jidx, ublk, iblk, ubb, ibb, phall,
               ubias, ibias, outv, semg0, semg1, semb0, semb1):
    _mf_body(users_hbm, items_hbm, ue_t, ie_t, ub_t, ib_t, out_hbm,
             uidx, iidx, jidx, ublk, iblk, ubb, ibb, phall,
             ubias, ibias, outv, semg0, semg1, semb0, semb1)


def kernel(users, items, user_emb, item_emb, user_bias, item_bias):
    return _mf_kernel(users.astype(jnp.int32), items.astype(jnp.int32),
                      user_emb.T, item_emb.T, user_bias.T, item_bias.T)
